# baseline (speedup 1.0000x reference)
import jax
import jax.numpy as jnp
from jax import lax
from jax.experimental import pallas as pl
from jax.experimental.pallas import tpu as pltpu
from jax.experimental.pallas import tpu_sc as plsc

_B = 64
_S = 512
_M = _S // 2
_IMG_W = 1440.0
_IMG_H = 2560.0
_NW = 32
_NG = 16
_GSEG = _M
_CAP = _NG * _GSEG


def _sc_body(id_hbm, pid_hbm, box_hbm, out_hbm,
             idr_v, pidr_v, box_v, idr2_v, pidr2_v, box2_v,
             idv_v, pidv_v, rep_v,
             x1_v, y1_v, x2_v, y2_v, ar_v, last_v,
             cnts_v, cur_v,
             gx1, gy1, gx2, gy2, gar, accrow_v, sem, sem2):
    cid = lax.axis_index("c")
    sid = lax.axis_index("s")
    wid = sid * 2 + cid
    iota = lax.iota(jnp.int32, 16)
    ones16 = jnp.ones((16,), jnp.int32)

    tot = jnp.zeros((16,), jnp.float32)
    cnt = jnp.zeros((16,), jnp.float32)
    b0 = wid * 2
    cps0 = [
        pltpu.async_copy(id_hbm.at[b0], idr_v, sem),
        pltpu.async_copy(pid_hbm.at[b0], pidr_v, sem),
        pltpu.async_copy(box_hbm.at[b0], box_v, sem),
    ]
    cps1 = [
        pltpu.async_copy(id_hbm.at[b0 + 1], idr2_v, sem2),
        pltpu.async_copy(pid_hbm.at[b0 + 1], pidr2_v, sem2),
        pltpu.async_copy(box_hbm.at[b0 + 1], box2_v, sem2),
    ]
    for bi, (cps, idr_v, pidr_v, box_v) in enumerate(
            [(cps0, idr_v, pidr_v, box_v), (cps1, idr2_v, pidr2_v, box2_v)]):
        for cp in cps:
            cp.wait()

        cnts_v[pl.ds(0, _NG)] = jnp.zeros((_NG,), jnp.int32)

        def chunk_b(c, carry):
            s = pl.ds(c * 16, 16)
            pos = c * 16 + iota
            oidx = 2 * pos + 1
            v = plsc.load_gather(idr_v, [oidx])
            p = plsc.load_gather(pidr_v, [oidx])
            idv_v[s] = v
            pidv_v[s] = p
            fidx = 4 * oidx
            cx = plsc.load_gather(box_v, [fidx]) * _IMG_W
            cy = plsc.load_gather(box_v, [fidx + 1]) * _IMG_H
            wp = plsc.load_gather(box_v, [fidx + 2]) * _IMG_W
            hp = plsc.load_gather(box_v, [fidx + 3]) * _IMG_H
            x1_v[s] = cx - wp * 0.5
            y1_v[s] = cy - hp * 0.5
            x2_v[s] = cx + wp * 0.5
            y2_v[s] = cy + hp * 0.5
            ar_v[s] = wp * hp
            _, lastm = plsc.scan_count(v)
            plsc.store_scatter(last_v, [v], pos, mask=lastm)
            return carry

        lax.fori_loop(0, 16, chunk_b, jnp.int32(0))

        def chunk_c1(c, carry):
            s = pl.ds(c * 16, 16)
            v = idv_v[s]
            pos = c * 16 + iota
            rep = plsc.load_gather(last_v, [v]) == pos
            rep_v[s] = rep.astype(jnp.int32)
            plsc.addupdate_scatter(cnts_v, [pidv_v[s]], ones16, mask=rep)
            return carry

        lax.fori_loop(0, 16, chunk_c1, jnp.int32(0))

        cur_v[...] = iota * _GSEG

        def chunk_c2(c, carry):
            s = pl.ds(c * 16, 16)
            rep = rep_v[s] != 0
            p = pidv_v[s]
            rank, _ = plsc.scan_count(p, mask=rep)
            base = plsc.load_gather(cur_v, [p])
            slot = base + rank - 1
            plsc.store_scatter(gx1, [slot], x1_v[s], mask=rep)
            plsc.store_scatter(gy1, [slot], y1_v[s], mask=rep)
            plsc.store_scatter(gx2, [slot], x2_v[s], mask=rep)
            plsc.store_scatter(gy2, [slot], y2_v[s], mask=rep)
            plsc.store_scatter(gar, [slot], ar_v[s], mask=rep)
            plsc.addupdate_scatter(cur_v, [p], ones16, mask=rep)
            return carry

        lax.fori_loop(0, 16, chunk_c2, jnp.int32(0))

        def group(g, carry):
            base = g * _GSEG
            ng = cnts_v[pl.ds(g, 16)][0]

            def small(carry2):
                s = pl.ds(base, 16)
                cx1 = gx1[s]
                cy1 = gy1[s]
                cx2 = gx2[s]
                cy2 = gy2[s]
                car = gar[s]

                mhi = iota < ng

                def one_row(r, tot3, cnt3):
                    ii = jnp.full((16,), base + r, jnp.int32)
                    rx1 = plsc.load_gather(gx1, [ii])
                    ry1 = plsc.load_gather(gy1, [ii])
                    rx2 = plsc.load_gather(gx2, [ii])
                    ry2 = plsc.load_gather(gy2, [ii])
                    rar = plsc.load_gather(gar, [ii])
                    m = (iota > r) & mhi
                    xl = jnp.maximum(rx1, cx1)
                    yt = jnp.maximum(ry1, cy1)
                    xr = jnp.minimum(rx2, cx2)
                    yb = jnp.minimum(ry2, cy2)
                    legal = (xr >= xl) & (yb >= yt) & m
                    inter = (xr - xl) * (yb - yt)
                    amin = jnp.minimum(rar, car)
                    iou = jnp.where(legal, inter / amin, jnp.float32(0.0))
                    return tot3 + iou, cnt3 + legal.astype(jnp.float32)

                def row2(c, carry3):
                    tot3, cnt3 = carry3
                    tot3, cnt3 = one_row(2 * c, tot3, cnt3)
                    tot3, cnt3 = one_row(2 * c + 1, tot3, cnt3)
                    return tot3, cnt3

                return lax.fori_loop(0, ng // 2, row2, carry2)

            def big(carry2):
                nch_hi = g * (_GSEG // 16) + (ng + 15) // 16
                en = base + ng

                def row(r, carry3):
                    i = base + r
                    ii = jnp.full((16,), i, jnp.int32)
                    rx1 = plsc.load_gather(gx1, [ii])
                    ry1 = plsc.load_gather(gy1, [ii])
                    rx2 = plsc.load_gather(gx2, [ii])
                    ry2 = plsc.load_gather(gy2, [ii])
                    rar = plsc.load_gather(gar, [ii])

                    def col_chunk(c, carry4):
                        tot4, cnt4 = carry4
                        s = pl.ds(c * 16, 16)
                        colidx = c * 16 + iota
                        m = (colidx > i) & (colidx < en)
                        xl = jnp.maximum(rx1, gx1[s])
                        yt = jnp.maximum(ry1, gy1[s])
                        xr = jnp.minimum(rx2, gx2[s])
                        yb = jnp.minimum(ry2, gy2[s])
                        legal = (xr >= xl) & (yb >= yt) & m
                        inter = (xr - xl) * (yb - yt)
                        amin = jnp.minimum(rar, gar[s])
                        iou = jnp.where(legal, inter / amin, jnp.float32(0.0))
                        return tot4 + iou, cnt4 + legal.astype(jnp.float32)

                    return lax.fori_loop(i // 16, nch_hi, col_chunk, carry3)

                return lax.fori_loop(0, ng - 1, row, carry2)

            return lax.cond(ng <= 16, small, big, carry)

        tot, cnt = lax.fori_loop(0, _NG, group, (tot, cnt))

    accrow_v[...] = tot
    pltpu.sync_copy(accrow_v, out_hbm.at[wid])
    accrow_v[...] = cnt
    pltpu.sync_copy(accrow_v, out_hbm.at[_NW + wid])


@jax.jit
def _sc_call(id_in, pid_in, box_in):
    mesh = plsc.VectorSubcoreMesh(core_axis_name="c", subcore_axis_name="s")
    run = pl.kernel(
        _sc_body,
        out_type=jax.ShapeDtypeStruct((2 * _NW, 16), jnp.float32),
        mesh=mesh,
        compiler_params=pltpu.CompilerParams(needs_layout_passes=False),
        scratch_types=[
            pltpu.VMEM((_S,), jnp.int32),
            pltpu.VMEM((_S,), jnp.int32),
            pltpu.VMEM((4 * _S,), jnp.float32),
            pltpu.VMEM((_S,), jnp.int32),
            pltpu.VMEM((_S,), jnp.int32),
            pltpu.VMEM((4 * _S,), jnp.float32),
            pltpu.VMEM((_M,), jnp.int32),
            pltpu.VMEM((_M,), jnp.int32),
            pltpu.VMEM((_M,), jnp.int32),
            pltpu.VMEM((_M,), jnp.float32),
            pltpu.VMEM((_M,), jnp.float32),
            pltpu.VMEM((_M,), jnp.float32),
            pltpu.VMEM((_M,), jnp.float32),
            pltpu.VMEM((_M,), jnp.float32),
            pltpu.VMEM((_M,), jnp.int32),
            pltpu.VMEM((2 * _NG,), jnp.int32),
            pltpu.VMEM((_NG,), jnp.int32),
            pltpu.VMEM((_CAP,), jnp.float32),
            pltpu.VMEM((_CAP,), jnp.float32),
            pltpu.VMEM((_CAP,), jnp.float32),
            pltpu.VMEM((_CAP,), jnp.float32),
            pltpu.VMEM((_CAP,), jnp.float32),
            pltpu.VMEM((16,), jnp.float32),
            pltpu.SemaphoreType.DMA,
            pltpu.SemaphoreType.DMA,
        ],
    )
    return run(id_in, pid_in, box_in)


def kernel(pred_boxes, id, parent_id, type_id):
    out = _sc_call(id, parent_id, pred_boxes.reshape(_B, 4 * _S))
    total = jnp.sum(out[:_NW])
    cntf = jnp.sum(out[_NW:])
    ratio = total / cntf
    bad = (cntf == 0) | jnp.logical_not(ratio >= 0.0) | jnp.logical_not(ratio <= 1.0)
    return jnp.where(bad, jnp.asarray(0.0, dtype=jnp.float32), total)


# speedup vs baseline: 1.0079x; 1.0079x over previous
import jax
import jax.numpy as jnp
from jax import lax
from jax.experimental import pallas as pl
from jax.experimental.pallas import tpu as pltpu
from jax.experimental.pallas import tpu_sc as plsc

_B = 64
_S = 512
_M = _S // 2
_IMG_W = 1440.0
_IMG_H = 2560.0
_NW = 32
_NG = 16
_GSEG = _M
_CAP = _NG * _GSEG


def _sc_body(id_hbm, pid_hbm, box_hbm, out_hbm,
             idr_v, pidr_v, box_v, idr2_v, pidr2_v, box2_v,
             idv_v, pidv_v, rep_v,
             x1_v, y1_v, x2_v, y2_v, ar_v, last_v,
             cnts_v, cur_v,
             gx1, gy1, gx2, gy2, gar, accrow_v, sem, sem2):
    cid = lax.axis_index("c")
    sid = lax.axis_index("s")
    wid = sid * 2 + cid
    iota = lax.iota(jnp.int32, 16)
    ones16 = jnp.ones((16,), jnp.int32)

    tot = jnp.zeros((16,), jnp.float32)
    cnt = jnp.zeros((16,), jnp.float32)
    b0 = wid * 2
    cps0 = [
        pltpu.async_copy(id_hbm.at[b0], idr_v, sem),
        pltpu.async_copy(pid_hbm.at[b0], pidr_v, sem),
        pltpu.async_copy(box_hbm.at[b0], box_v, sem),
    ]
    cps1 = [
        pltpu.async_copy(id_hbm.at[b0 + 1], idr2_v, sem2),
        pltpu.async_copy(pid_hbm.at[b0 + 1], pidr2_v, sem2),
        pltpu.async_copy(box_hbm.at[b0 + 1], box2_v, sem2),
    ]
    for bi, (cps, idr_v, pidr_v, box_v) in enumerate(
            [(cps0, idr_v, pidr_v, box_v), (cps1, idr2_v, pidr2_v, box2_v)]):
        for cp in cps:
            cp.wait()

        cnts_v[pl.ds(0, _NG)] = jnp.zeros((_NG,), jnp.int32)

        def chunk_b(c, carry):
            s = pl.ds(c * 16, 16)
            pos = c * 16 + iota
            oidx = 2 * pos + 1
            v = plsc.load_gather(idr_v, [oidx])
            p = plsc.load_gather(pidr_v, [oidx])
            idv_v[s] = v
            pidv_v[s] = p
            fidx = 4 * oidx
            cx = plsc.load_gather(box_v, [fidx]) * _IMG_W
            cy = plsc.load_gather(box_v, [fidx + 1]) * _IMG_H
            wp = plsc.load_gather(box_v, [fidx + 2]) * _IMG_W
            hp = plsc.load_gather(box_v, [fidx + 3]) * _IMG_H
            x1_v[s] = cx - wp * 0.5
            y1_v[s] = cy - hp * 0.5
            x2_v[s] = cx + wp * 0.5
            y2_v[s] = cy + hp * 0.5
            ar_v[s] = wp * hp
            _, lastm = plsc.scan_count(v)
            plsc.store_scatter(last_v, [v], pos, mask=lastm)
            return carry

        lax.fori_loop(0, 16, chunk_b, jnp.int32(0))

        def chunk_c1(c, carry):
            s = pl.ds(c * 16, 16)
            v = idv_v[s]
            pos = c * 16 + iota
            rep = plsc.load_gather(last_v, [v]) == pos
            rep_v[s] = rep.astype(jnp.int32)
            plsc.addupdate_scatter(cnts_v, [pidv_v[s]], ones16, mask=rep)
            return carry

        lax.fori_loop(0, 16, chunk_c1, jnp.int32(0))

        cur_v[...] = iota * _GSEG

        def chunk_c2(c, carry):
            s = pl.ds(c * 16, 16)
            rep = rep_v[s] != 0
            p = pidv_v[s]
            rank, _ = plsc.scan_count(p, mask=rep)
            base = plsc.load_gather(cur_v, [p])
            slot = base + rank - 1
            plsc.store_scatter(gx1, [slot], x1_v[s], mask=rep)
            plsc.store_scatter(gy1, [slot], y1_v[s], mask=rep)
            plsc.store_scatter(gx2, [slot], x2_v[s], mask=rep)
            plsc.store_scatter(gy2, [slot], y2_v[s], mask=rep)
            plsc.store_scatter(gar, [slot], ar_v[s], mask=rep)
            plsc.addupdate_scatter(cur_v, [p], ones16, mask=rep)
            return carry

        lax.fori_loop(0, 16, chunk_c2, jnp.int32(0))

        def group(g, carry):
            base = g * _GSEG
            ng = cnts_v[pl.ds(g, 16)][0]

            def small(carry2):
                s = pl.ds(base, 16)
                cx1 = gx1[s]
                cy1 = gy1[s]
                cx2 = gx2[s]
                cy2 = gy2[s]
                car = gar[s]

                mhi = iota < ng

                def row(r, carry3):
                    tot3, cnt3 = carry3
                    ii = jnp.full((16,), base + r, jnp.int32)
                    rx1 = plsc.load_gather(gx1, [ii])
                    ry1 = plsc.load_gather(gy1, [ii])
                    rx2 = plsc.load_gather(gx2, [ii])
                    ry2 = plsc.load_gather(gy2, [ii])
                    rar = plsc.load_gather(gar, [ii])
                    m = (iota > r) & mhi
                    xl = jnp.maximum(rx1, cx1)
                    yt = jnp.maximum(ry1, cy1)
                    xr = jnp.minimum(rx2, cx2)
                    yb = jnp.minimum(ry2, cy2)
                    legal = (xr >= xl) & (yb >= yt) & m
                    inter = (xr - xl) * (yb - yt)
                    amin = jnp.minimum(rar, car)
                    iou = jnp.where(legal, inter / amin, jnp.float32(0.0))
                    return tot3 + iou, cnt3 + legal.astype(jnp.float32)

                return lax.fori_loop(0, ng - 1, row, carry2)

            def big(carry2):
                nch_hi = g * (_GSEG // 16) + (ng + 15) // 16
                en = base + ng

                def row(r, carry3):
                    i = base + r
                    ii = jnp.full((16,), i, jnp.int32)
                    rx1 = plsc.load_gather(gx1, [ii])
                    ry1 = plsc.load_gather(gy1, [ii])
                    rx2 = plsc.load_gather(gx2, [ii])
                    ry2 = plsc.load_gather(gy2, [ii])
                    rar = plsc.load_gather(gar, [ii])

                    def col_chunk(c, carry4):
                        tot4, cnt4 = carry4
                        s = pl.ds(c * 16, 16)
                        colidx = c * 16 + iota
                        m = (colidx > i) & (colidx < en)
                        xl = jnp.maximum(rx1, gx1[s])
                        yt = jnp.maximum(ry1, gy1[s])
                        xr = jnp.minimum(rx2, gx2[s])
                        yb = jnp.minimum(ry2, gy2[s])
                        legal = (xr >= xl) & (yb >= yt) & m
                        inter = (xr - xl) * (yb - yt)
                        amin = jnp.minimum(rar, gar[s])
                        iou = jnp.where(legal, inter / amin, jnp.float32(0.0))
                        return tot4 + iou, cnt4 + legal.astype(jnp.float32)

                    return lax.fori_loop(i // 16, nch_hi, col_chunk, carry3)

                return lax.fori_loop(0, ng - 1, row, carry2)

            return lax.cond(ng <= 16, small, big, carry)

        tot, cnt = lax.fori_loop(0, _NG, group, (tot, cnt))

    accrow_v[...] = tot
    pltpu.sync_copy(accrow_v, out_hbm.at[wid])
    accrow_v[...] = cnt
    pltpu.sync_copy(accrow_v, out_hbm.at[_NW + wid])


@jax.jit
def _sc_call(id_in, pid_in, box_in):
    mesh = plsc.VectorSubcoreMesh(core_axis_name="c", subcore_axis_name="s")
    run = pl.kernel(
        _sc_body,
        out_type=jax.ShapeDtypeStruct((2 * _NW, 16), jnp.float32),
        mesh=mesh,
        compiler_params=pltpu.CompilerParams(needs_layout_passes=False),
        scratch_types=[
            pltpu.VMEM((_S,), jnp.int32),
            pltpu.VMEM((_S,), jnp.int32),
            pltpu.VMEM((4 * _S,), jnp.float32),
            pltpu.VMEM((_S,), jnp.int32),
            pltpu.VMEM((_S,), jnp.int32),
            pltpu.VMEM((4 * _S,), jnp.float32),
            pltpu.VMEM((_M,), jnp.int32),
            pltpu.VMEM((_M,), jnp.int32),
            pltpu.VMEM((_M,), jnp.int32),
            pltpu.VMEM((_M,), jnp.float32),
            pltpu.VMEM((_M,), jnp.float32),
            pltpu.VMEM((_M,), jnp.float32),
            pltpu.VMEM((_M,), jnp.float32),
            pltpu.VMEM((_M,), jnp.float32),
            pltpu.VMEM((_M,), jnp.int32),
            pltpu.VMEM((2 * _NG,), jnp.int32),
            pltpu.VMEM((_NG,), jnp.int32),
            pltpu.VMEM((_CAP,), jnp.float32),
            pltpu.VMEM((_CAP,), jnp.float32),
            pltpu.VMEM((_CAP,), jnp.float32),
            pltpu.VMEM((_CAP,), jnp.float32),
            pltpu.VMEM((_CAP,), jnp.float32),
            pltpu.VMEM((16,), jnp.float32),
            pltpu.SemaphoreType.DMA,
            pltpu.SemaphoreType.DMA,
        ],
    )
    return run(id_in, pid_in, box_in)


def kernel(pred_boxes, id, parent_id, type_id):
    out = _sc_call(id, parent_id, pred_boxes.reshape(_B, 4 * _S))
    total = jnp.sum(out[:_NW])
    cntf = jnp.sum(out[_NW:])
    ratio = total / cntf
    bad = (cntf == 0) | jnp.logical_not(ratio >= 0.0) | jnp.logical_not(ratio <= 1.0)
    return jnp.where(bad, jnp.asarray(0.0, dtype=jnp.float32), total)
